# baseline (speedup 1.0000x reference)
import ml_dtypes
import numpy as np
import jax
import jax.numpy as jnp
from jax import lax
from jax.experimental import pallas as pl
from jax.experimental.pallas import tpu as pltpu

_B = 256


def _band(nout, nin):
    b = np.zeros((2, 5, nout, nin), np.float32)
    for p in range(2):
        for k in range(5):
            for y in range(nout):
                b[p, k, y, 2 * y + p + k] = 1.0
    return b


_BAND1 = _band(12, 28)
_BAND2 = _band(4, 12)

_KB1 = np.zeros((26, 4, 144, 785), np.float32)
_KB1[:25, :, :, :784] = np.einsum(
    'pkyi,qlxj->klpqyxij', _BAND1, _BAND1).reshape(25, 4, 144, 784)
_KB1[25, :, :, 784] = 1.0
_KB1 = _KB1.reshape(26, 4 * 144 * 785).astype(ml_dtypes.bfloat16)
_KB2 = np.einsum('pkyi,qlxj->klpqyxij', _BAND2, _BAND2).reshape(
    25, 4 * 16 * 144).astype(ml_dtypes.bfloat16)


def _lenet_body(xb_ref, w1a_ref, w2a_ref, w1f_ref, w2f_ref, w3f_ref, out_ref):
    f32 = jnp.float32
    bf16 = jnp.bfloat16
    ones_col = jnp.ones((_B, 1), f32)
    ones_row = jnp.ones((1, _B), f32)

    xaug = jnp.concatenate([xb_ref[...], ones_col], axis=1).astype(bf16)
    h1 = lax.dot_general(w1a_ref[...], xaug, (((1,), (1,)), ((), ())),
                         preferred_element_type=f32)
    h1 = jnp.maximum(h1, 0.0)

    p1 = jnp.concatenate(
        [jnp.maximum(
            jnp.maximum(h1[(c * 4 + 0) * 144:(c * 4 + 1) * 144],
                        h1[(c * 4 + 1) * 144:(c * 4 + 2) * 144]),
            jnp.maximum(h1[(c * 4 + 2) * 144:(c * 4 + 3) * 144],
                        h1[(c * 4 + 3) * 144:(c * 4 + 4) * 144]))
         for c in range(6)] + [ones_row], axis=0).astype(bf16)

    h2 = jnp.dot(w2a_ref[...], p1, preferred_element_type=f32)
    h2 = jnp.maximum(h2, 0.0)

    p2 = jnp.concatenate(
        [jnp.maximum(
            jnp.maximum(h2[(c * 4 + 0) * 16:(c * 4 + 1) * 16],
                        h2[(c * 4 + 1) * 16:(c * 4 + 2) * 16]),
            jnp.maximum(h2[(c * 4 + 2) * 16:(c * 4 + 3) * 16],
                        h2[(c * 4 + 3) * 16:(c * 4 + 4) * 16]))
         for c in range(12)] + [ones_row], axis=0)

    h3 = jnp.maximum(jnp.dot(w1f_ref[...], p2, preferred_element_type=f32), 0.0)
    h3 = jnp.concatenate([h3, ones_row], axis=0)
    h4 = jnp.maximum(jnp.dot(w2f_ref[...], h3, preferred_element_type=f32), 0.0)
    h4 = jnp.concatenate([h4, ones_row], axis=0)
    out_ref[...] = lax.dot_general(h4, w3f_ref[...], (((0,), (1,)), ((), ())),
                                   preferred_element_type=f32)


def kernel(x, conv1_w, conv1_b, conv2_w, conv2_b,
           fc1_w, fc1_b, fc2_w, fc2_b, out_w, out_b):
    f32 = jnp.float32
    bf16 = jnp.bfloat16
    n = x.shape[0]
    nt = -(-n // _B)
    npad = nt * _B

    x2 = x.astype(f32).reshape(n, 784)
    if npad != n:
        x2 = jnp.pad(x2, ((0, npad - n), (0, 0)))

    w1s = jnp.concatenate(
        [conv1_w.astype(f32).reshape(6, 25), conv1_b.astype(f32)[:, None]],
        axis=1)
    w1s8 = jnp.pad(w1s, ((0, 2), (0, 0))).astype(bf16)
    w1a = jnp.dot(w1s8, jnp.asarray(_KB1),
                  preferred_element_type=f32).astype(bf16).reshape(4608, 785)
    w2core = jnp.dot(conv2_w.astype(bf16).reshape(72, 25), jnp.asarray(_KB2),
                     preferred_element_type=f32)
    w2core = jnp.transpose(w2core.reshape(12, 6, 64, 144),
                           (0, 2, 1, 3)).reshape(768, 864)
    w2a = jnp.concatenate(
        [w2core, jnp.repeat(conv2_b.astype(f32), 64)[:, None]],
        axis=1).astype(bf16)

    w1f = jnp.concatenate([fc1_w.astype(f32), fc1_b.astype(f32)[:, None]], axis=1)
    w2f = jnp.concatenate([fc2_w.astype(f32), fc2_b.astype(f32)[:, None]], axis=1)
    w3f = jnp.concatenate([out_w.astype(f32), out_b.astype(f32)[:, None]], axis=1)

    def whole(a):
        return pl.BlockSpec(a.shape, lambda i: (0,) * a.ndim)

    out = pl.pallas_call(
        _lenet_body,
        out_shape=jax.ShapeDtypeStruct((npad, 10), f32),
        grid=(nt,),
        in_specs=[
            pl.BlockSpec((_B, 784), lambda i: (i, 0)),
            whole(w1a), whole(w2a), whole(w1f), whole(w2f), whole(w3f),
        ],
        out_specs=pl.BlockSpec((_B, 10), lambda i: (i, 0)),
        compiler_params=pltpu.CompilerParams(
            dimension_semantics=("parallel",),
            vmem_limit_bytes=48 * 1024 * 1024,
        ),
    )(x2, w1a, w2a, w1f, w2f, w3f)

    return out[:n]


# speedup vs baseline: 1.2719x; 1.2719x over previous
import numpy as np
import jax
import jax.numpy as jnp
from jax import lax
from jax.experimental import pallas as pl
from jax.experimental.pallas import tpu as pltpu

_B = 256


def _masks1():
    e = np.zeros((2, 25, 12, 140), np.float32)
    for q in range(2):
        for k in range(5):
            for l in range(5):
                for x in range(12):
                    e[q, k * 5 + l, x, k * 28 + 2 * x + q + l] = 1.0
    return e


def _masks2():
    e = np.zeros((2, 25, 4, 60), np.float32)
    for q in range(2):
        for k in range(5):
            for l in range(5):
                for x in range(4):
                    e[q, k * 5 + l, x, k * 12 + 2 * x + q + l] = 1.0
    return e


_E1 = _masks1()
_E2 = _masks2()


def _build_weights(w1_ref, b1_ref, w2_ref, b2_ref, e1_ref, e2_ref,
                   w1a_ref, w2a_ref):
    f32 = jnp.float32
    bf16 = jnp.bfloat16

    w1a_ref[...] = jnp.zeros(w1a_ref.shape, bf16)
    w2a_ref[...] = jnp.zeros(w2a_ref.shape, bf16)

    for c in range(6):
        for q in range(2):
            s = sum(w1_ref[c * 25 + t] * e1_ref[q, t] for t in range(25))
            sw = s.astype(bf16)
            for p in range(2):
                base = (c * 4 + p * 2 + q) * 144
                for y in range(12):
                    col = (2 * y + p) * 28
                    w1a_ref[pl.ds(base + y * 12, 12), col:col + 140] = sw
        w1a_ref[pl.ds(c * 576, 576), 784:785] = jnp.full(
            (576, 1), b1_ref[c], bf16)

    for co in range(12):
        for cin in range(6):
            for q in range(2):
                s = sum(w2_ref[(co * 6 + cin) * 25 + t] * e2_ref[q, t]
                        for t in range(25))
                sw = s.astype(bf16)
                for p in range(2):
                    base = (co * 4 + p * 2 + q) * 16
                    for y in range(4):
                        col = cin * 144 + (2 * y + p) * 12
                        w2a_ref[pl.ds(base + y * 4, 4), col:col + 60] = sw
        w2a_ref[pl.ds(co * 64, 64), 864:865] = jnp.full(
            (64, 1), b2_ref[co], bf16)


def _lenet_body(xb_ref, w1_ref, b1_ref, w2_ref, b2_ref, e1_ref, e2_ref,
                w1f_ref, w2f_ref, w3f_ref, out_ref, w1a_ref, w2a_ref):
    f32 = jnp.float32
    bf16 = jnp.bfloat16

    @pl.when(pl.program_id(0) == 0)
    def _():
        _build_weights(w1_ref, b1_ref, w2_ref, b2_ref, e1_ref, e2_ref,
                       w1a_ref, w2a_ref)

    ones_col = jnp.ones((_B, 1), f32)
    ones_row = jnp.ones((1, _B), f32)

    xaug = jnp.concatenate([xb_ref[...], ones_col], axis=1).astype(bf16)
    h1 = lax.dot_general(w1a_ref[...], xaug, (((1,), (1,)), ((), ())),
                         preferred_element_type=f32)
    h1 = jnp.maximum(h1, 0.0)

    p1 = jnp.concatenate(
        [jnp.maximum(
            jnp.maximum(h1[(c * 4 + 0) * 144:(c * 4 + 1) * 144],
                        h1[(c * 4 + 1) * 144:(c * 4 + 2) * 144]),
            jnp.maximum(h1[(c * 4 + 2) * 144:(c * 4 + 3) * 144],
                        h1[(c * 4 + 3) * 144:(c * 4 + 4) * 144]))
         for c in range(6)] + [ones_row], axis=0).astype(bf16)

    h2 = jnp.dot(w2a_ref[...], p1, preferred_element_type=f32)
    h2 = jnp.maximum(h2, 0.0)

    p2 = jnp.concatenate(
        [jnp.maximum(
            jnp.maximum(h2[(c * 4 + 0) * 16:(c * 4 + 1) * 16],
                        h2[(c * 4 + 1) * 16:(c * 4 + 2) * 16]),
            jnp.maximum(h2[(c * 4 + 2) * 16:(c * 4 + 3) * 16],
                        h2[(c * 4 + 3) * 16:(c * 4 + 4) * 16]))
         for c in range(12)] + [ones_row], axis=0)

    h3 = jnp.maximum(jnp.dot(w1f_ref[...], p2, preferred_element_type=f32), 0.0)
    h3 = jnp.concatenate([h3, ones_row], axis=0)
    h4 = jnp.maximum(jnp.dot(w2f_ref[...], h3, preferred_element_type=f32), 0.0)
    h4 = jnp.concatenate([h4, ones_row], axis=0)
    out_ref[...] = lax.dot_general(h4, w3f_ref[...], (((0,), (1,)), ((), ())),
                                   preferred_element_type=f32)


def kernel(x, conv1_w, conv1_b, conv2_w, conv2_b,
           fc1_w, fc1_b, fc2_w, fc2_b, out_w, out_b):
    f32 = jnp.float32
    n = x.shape[0]
    nt = -(-n // _B)
    npad = nt * _B

    x2 = x.astype(f32).reshape(n, 784)
    if npad != n:
        x2 = jnp.pad(x2, ((0, npad - n), (0, 0)))

    w1s = conv1_w.astype(f32).reshape(150)
    w2s = conv2_w.astype(f32).reshape(1800)
    b1s = conv1_b.astype(f32)
    b2s = conv2_b.astype(f32)

    w1f = jnp.concatenate([fc1_w.astype(f32), fc1_b.astype(f32)[:, None]], axis=1)
    w2f = jnp.concatenate([fc2_w.astype(f32), fc2_b.astype(f32)[:, None]], axis=1)
    w3f = jnp.concatenate([out_w.astype(f32), out_b.astype(f32)[:, None]], axis=1)

    smem = pl.BlockSpec(memory_space=pltpu.MemorySpace.SMEM)

    def whole(a):
        return pl.BlockSpec(a.shape, lambda i: (0,) * a.ndim)

    e1 = jnp.asarray(_E1)
    e2 = jnp.asarray(_E2)

    out = pl.pallas_call(
        _lenet_body,
        out_shape=jax.ShapeDtypeStruct((npad, 10), f32),
        grid=(nt,),
        in_specs=[
            pl.BlockSpec((_B, 784), lambda i: (i, 0)),
            smem, smem, smem, smem,
            whole(e1), whole(e2),
            whole(w1f), whole(w2f), whole(w3f),
        ],
        out_specs=pl.BlockSpec((_B, 10), lambda i: (i, 0)),
        scratch_shapes=[
            pltpu.VMEM((3456, 785), jnp.bfloat16),
            pltpu.VMEM((768, 865), jnp.bfloat16),
        ],
        compiler_params=pltpu.CompilerParams(
            dimension_semantics=("arbitrary",),
            vmem_limit_bytes=48 * 1024 * 1024,
        ),
    )(x2, w1s, b1s, w2s, b2s, e1, e2, w1f, w2f, w3f)

    return out[:n]


# speedup vs baseline: 1.4159x; 1.1132x over previous
import numpy as np
import jax
import jax.numpy as jnp
from jax import lax
from jax.experimental import pallas as pl
from jax.experimental.pallas import tpu as pltpu

_B = 256


def _masks1():
    e = np.zeros((2, 25, 12, 140), np.float32)
    for q in range(2):
        for k in range(5):
            for l in range(5):
                for x in range(12):
                    e[q, k * 5 + l, x, k * 28 + 2 * x + q + l] = 1.0
    return e


def _masks2():
    e = np.zeros((2, 25, 4, 60), np.float32)
    for q in range(2):
        for k in range(5):
            for l in range(5):
                for x in range(4):
                    e[q, k * 5 + l, x, k * 12 + 2 * x + q + l] = 1.0
    return e


_E1 = _masks1()
_E2 = _masks2()


def _build_weights(w1_ref, b1_ref, w2_ref, b2_ref, e1_ref, e2_ref,
                   w1a_ref, w2a_ref):
    f32 = jnp.float32
    bf16 = jnp.bfloat16

    w1a_ref[...] = jnp.zeros(w1a_ref.shape, bf16)
    w2a_ref[...] = jnp.zeros(w2a_ref.shape, bf16)

    for c in range(6):
        for q in range(2):
            s = sum(w1_ref[c * 25 + t] * e1_ref[q, t] for t in range(25))
            sw = s.astype(bf16)
            for p in range(2):
                base = (c * 4 + p * 2 + q) * 144
                for y in range(12):
                    col = (2 * y + p) * 28
                    w1a_ref[pl.ds(base + y * 12, 12), col:col + 140] = sw
        w1a_ref[pl.ds(c * 576, 576), 784:785] = jnp.full(
            (576, 1), b1_ref[c], bf16)

    for co in range(12):
        for cin in range(6):
            for q in range(2):
                s = sum(w2_ref[(co * 6 + cin) * 25 + t] * e2_ref[q, t]
                        for t in range(25))
                sw = s.astype(bf16)
                for p in range(2):
                    base = (co * 4 + p * 2 + q) * 16
                    for y in range(4):
                        col = cin * 144 + (2 * y + p) * 12
                        w2a_ref[pl.ds(base + y * 4, 4), col:col + 60] = sw
        w2a_ref[pl.ds(co * 64, 64), 864:865] = jnp.full(
            (64, 1), b2_ref[co], bf16)


def _lenet_body(xb_ref, w1_ref, b1_ref, w2_ref, b2_ref, e1_ref, e2_ref,
                w1f_ref, w2f_ref, w3f_ref, out_ref, w1a_ref, w2a_ref):
    f32 = jnp.float32
    bf16 = jnp.bfloat16

    @pl.when(pl.program_id(0) == 0)
    def _():
        _build_weights(w1_ref, b1_ref, w2_ref, b2_ref, e1_ref, e2_ref,
                       w1a_ref, w2a_ref)

    ones_row = jnp.ones((1, _B), f32)

    xaug = jnp.concatenate([xb_ref[...], ones_row], axis=0).astype(bf16)
    h1 = jnp.dot(w1a_ref[...], xaug, preferred_element_type=f32)
    h1 = jnp.maximum(h1, 0.0)

    p1 = jnp.concatenate(
        [jnp.maximum(
            jnp.maximum(h1[(c * 4 + 0) * 144:(c * 4 + 1) * 144],
                        h1[(c * 4 + 1) * 144:(c * 4 + 2) * 144]),
            jnp.maximum(h1[(c * 4 + 2) * 144:(c * 4 + 3) * 144],
                        h1[(c * 4 + 3) * 144:(c * 4 + 4) * 144]))
         for c in range(6)] + [ones_row], axis=0).astype(bf16)

    h2 = jnp.dot(w2a_ref[...], p1, preferred_element_type=f32)
    h2 = jnp.maximum(h2, 0.0)

    p2 = jnp.concatenate(
        [jnp.maximum(
            jnp.maximum(h2[(c * 4 + 0) * 16:(c * 4 + 1) * 16],
                        h2[(c * 4 + 1) * 16:(c * 4 + 2) * 16]),
            jnp.maximum(h2[(c * 4 + 2) * 16:(c * 4 + 3) * 16],
                        h2[(c * 4 + 3) * 16:(c * 4 + 4) * 16]))
         for c in range(12)] + [ones_row], axis=0)

    h3 = jnp.maximum(jnp.dot(w1f_ref[...], p2, preferred_element_type=f32), 0.0)
    h3 = jnp.concatenate([h3, ones_row], axis=0)
    h4 = jnp.maximum(jnp.dot(w2f_ref[...], h3, preferred_element_type=f32), 0.0)
    h4 = jnp.concatenate([h4, ones_row], axis=0)
    out_ref[...] = lax.dot_general(h4, w3f_ref[...], (((0,), (1,)), ((), ())),
                                   preferred_element_type=f32)


def kernel(x, conv1_w, conv1_b, conv2_w, conv2_b,
           fc1_w, fc1_b, fc2_w, fc2_b, out_w, out_b):
    f32 = jnp.float32
    n = x.shape[0]
    nt = -(-n // _B)
    npad = nt * _B

    x2 = x.astype(f32).reshape(n, 784).T
    if npad != n:
        x2 = jnp.pad(x2, ((0, 0), (0, npad - n)))

    w1s = conv1_w.astype(f32).reshape(150)
    w2s = conv2_w.astype(f32).reshape(1800)
    b1s = conv1_b.astype(f32)
    b2s = conv2_b.astype(f32)

    w1f = jnp.concatenate([fc1_w.astype(f32), fc1_b.astype(f32)[:, None]], axis=1)
    w2f = jnp.concatenate([fc2_w.astype(f32), fc2_b.astype(f32)[:, None]], axis=1)
    w3f = jnp.concatenate([out_w.astype(f32), out_b.astype(f32)[:, None]], axis=1)

    smem = pl.BlockSpec(memory_space=pltpu.MemorySpace.SMEM)

    def whole(a):
        return pl.BlockSpec(a.shape, lambda i: (0,) * a.ndim)

    e1 = jnp.asarray(_E1)
    e2 = jnp.asarray(_E2)

    out = pl.pallas_call(
        _lenet_body,
        out_shape=jax.ShapeDtypeStruct((npad, 10), f32),
        grid=(nt,),
        in_specs=[
            pl.BlockSpec((784, _B), lambda i: (0, i)),
            smem, smem, smem, smem,
            whole(e1), whole(e2),
            whole(w1f), whole(w2f), whole(w3f),
        ],
        out_specs=pl.BlockSpec((_B, 10), lambda i: (i, 0)),
        scratch_shapes=[
            pltpu.VMEM((3456, 785), jnp.bfloat16),
            pltpu.VMEM((768, 865), jnp.bfloat16),
        ],
        compiler_params=pltpu.CompilerParams(
            dimension_semantics=("arbitrary",),
            vmem_limit_bytes=48 * 1024 * 1024,
        ),
    )(x2, w1s, b1s, w2s, b2s, e1, e2, w1f, w2f, w3f)

    return out[:n]


# speedup vs baseline: 1.7249x; 1.2182x over previous
import numpy as np
import jax
import jax.numpy as jnp
from jax import lax
from jax.experimental import pallas as pl
from jax.experimental.pallas import tpu as pltpu

_B = 256


def _masks1():
    e = np.zeros((2, 25, 12, 140), np.float32)
    for q in range(2):
        for k in range(5):
            for l in range(5):
                for x in range(12):
                    e[q, k * 5 + l, x, k * 28 + 2 * x + q + l] = 1.0
    return e


def _masks2():
    e = np.zeros((2, 25, 4, 60), np.float32)
    for q in range(2):
        for k in range(5):
            for l in range(5):
                for x in range(4):
                    e[q, k * 5 + l, x, k * 12 + 2 * x + q + l] = 1.0
    return e


_E1 = _masks1()
_E2 = _masks2()


def _build_weights(w1_ref, b1_ref, w2_ref, b2_ref, e1_ref, e2_ref,
                   w1a_ref, w2a_ref):
    f32 = jnp.float32
    bf16 = jnp.bfloat16

    w1a_ref[...] = jnp.zeros(w1a_ref.shape, bf16)
    w2a_ref[...] = jnp.zeros(w2a_ref.shape, bf16)

    for c in range(6):
        for q in range(2):
            s = sum(w1_ref[c * 25 + t] * e1_ref[q, t] for t in range(25))
            sw = s.astype(bf16)
            for p in range(2):
                base = (c * 4 + p * 2 + q) * 144
                for y in range(12):
                    col = (2 * y + p) * 28
                    w1a_ref[pl.ds(base + y * 12, 12), col:col + 140] = sw
        w1a_ref[pl.ds(c * 576, 576), 784:785] = jnp.full(
            (576, 1), b1_ref[c], bf16)

    for co in range(12):
        for cin in range(6):
            for q in range(2):
                s = sum(w2_ref[(co * 6 + cin) * 25 + t] * e2_ref[q, t]
                        for t in range(25))
                sw = s.astype(bf16)
                for p in range(2):
                    base = (co * 4 + p * 2 + q) * 16
                    for y in range(4):
                        col = cin * 144 + (2 * y + p) * 12
                        w2a_ref[pl.ds(base + y * 4, 4), col:col + 60] = sw
        w2a_ref[pl.ds(co * 64, 64), 864:865] = jnp.full(
            (64, 1), b2_ref[co], bf16)


def _lenet_body(xb_ref, w1_ref, b1_ref, w2_ref, b2_ref, e1_ref, e2_ref,
                w1f_ref, w2f_ref, w3f_ref, out_ref, w1a_ref, w2a_ref):
    f32 = jnp.float32
    bf16 = jnp.bfloat16

    @pl.when(pl.program_id(0) == 0)
    def _():
        _build_weights(w1_ref, b1_ref, w2_ref, b2_ref, e1_ref, e2_ref,
                       w1a_ref, w2a_ref)

    ones_row = jnp.ones((1, _B), f32)

    xaug = jnp.concatenate([xb_ref[...], ones_row], axis=0).astype(bf16)
    h1 = jnp.dot(w1a_ref[...], xaug, preferred_element_type=f32)
    h1 = jnp.maximum(h1, 0.0)

    p1 = jnp.concatenate(
        [jnp.maximum(
            jnp.maximum(h1[(c * 4 + 0) * 144:(c * 4 + 1) * 144],
                        h1[(c * 4 + 1) * 144:(c * 4 + 2) * 144]),
            jnp.maximum(h1[(c * 4 + 2) * 144:(c * 4 + 3) * 144],
                        h1[(c * 4 + 3) * 144:(c * 4 + 4) * 144]))
         for c in range(6)] + [ones_row], axis=0).astype(bf16)

    h2 = jnp.dot(w2a_ref[...], p1, preferred_element_type=f32)
    h2 = jnp.maximum(h2, 0.0)

    p2 = jnp.concatenate(
        [jnp.maximum(
            jnp.maximum(h2[(c * 4 + 0) * 16:(c * 4 + 1) * 16],
                        h2[(c * 4 + 1) * 16:(c * 4 + 2) * 16]),
            jnp.maximum(h2[(c * 4 + 2) * 16:(c * 4 + 3) * 16],
                        h2[(c * 4 + 3) * 16:(c * 4 + 4) * 16]))
         for c in range(12)] + [ones_row], axis=0)

    h3 = jnp.maximum(jnp.dot(w1f_ref[...], p2, preferred_element_type=f32), 0.0)
    h3 = jnp.concatenate([h3, ones_row], axis=0)
    h4 = jnp.maximum(jnp.dot(w2f_ref[...], h3, preferred_element_type=f32), 0.0)
    h4 = jnp.concatenate([h4, ones_row], axis=0)
    out_ref[...] = lax.dot_general(h4, w3f_ref[...], (((0,), (1,)), ((), ())),
                                   preferred_element_type=f32)


def kernel(x, conv1_w, conv1_b, conv2_w, conv2_b,
           fc1_w, fc1_b, fc2_w, fc2_b, out_w, out_b):
    f32 = jnp.float32
    n = x.shape[0]
    nt = -(-n // _B)
    npad = nt * _B

    x2 = x.astype(f32)[:, 0].transpose(1, 2, 0).reshape(784, n)
    if npad != n:
        x2 = jnp.pad(x2, ((0, 0), (0, npad - n)))

    w1s = conv1_w.astype(f32).reshape(150)
    w2s = conv2_w.astype(f32).reshape(1800)
    b1s = conv1_b.astype(f32)
    b2s = conv2_b.astype(f32)

    w1f = jnp.concatenate([fc1_w.astype(f32), fc1_b.astype(f32)[:, None]], axis=1)
    w2f = jnp.concatenate([fc2_w.astype(f32), fc2_b.astype(f32)[:, None]], axis=1)
    w3f = jnp.concatenate([out_w.astype(f32), out_b.astype(f32)[:, None]], axis=1)

    smem = pl.BlockSpec(memory_space=pltpu.MemorySpace.SMEM)

    def whole(a):
        return pl.BlockSpec(a.shape, lambda i: (0,) * a.ndim)

    e1 = jnp.asarray(_E1)
    e2 = jnp.asarray(_E2)

    out = pl.pallas_call(
        _lenet_body,
        out_shape=jax.ShapeDtypeStruct((npad, 10), f32),
        grid=(nt,),
        in_specs=[
            pl.BlockSpec((784, _B), lambda i: (0, i)),
            smem, smem, smem, smem,
            whole(e1), whole(e2),
            whole(w1f), whole(w2f), whole(w3f),
        ],
        out_specs=pl.BlockSpec((_B, 10), lambda i: (i, 0)),
        scratch_shapes=[
            pltpu.VMEM((3456, 785), jnp.bfloat16),
            pltpu.VMEM((768, 865), jnp.bfloat16),
        ],
        compiler_params=pltpu.CompilerParams(
            dimension_semantics=("arbitrary",),
            vmem_limit_bytes=48 * 1024 * 1024,
        ),
    )(x2, w1s, b1s, w2s, b2s, e1, e2, w1f, w2f, w3f)

    return out[:n]


# speedup vs baseline: 2.0498x; 1.1884x over previous
import numpy as np
import jax
import jax.numpy as jnp
from jax import lax
from jax.experimental import pallas as pl
from jax.experimental.pallas import tpu as pltpu

_B = 512


def _masks1():
    e = np.zeros((2, 25, 12, 140), np.float32)
    for q in range(2):
        for k in range(5):
            for l in range(5):
                for x in range(12):
                    e[q, k * 5 + l, x, k * 28 + 2 * x + q + l] = 1.0
    return e


def _masks2():
    e = np.zeros((2, 25, 4, 60), np.float32)
    for q in range(2):
        for k in range(5):
            for l in range(5):
                for x in range(4):
                    e[q, k * 5 + l, x, k * 12 + 2 * x + q + l] = 1.0
    return e


_E1 = _masks1()
_E2 = _masks2()


def _build_weights(w1_ref, b1_ref, w2_ref, b2_ref, e1_ref, e2_ref,
                   w1a_ref, w2a_ref):
    f32 = jnp.float32
    bf16 = jnp.bfloat16

    w1a_ref[...] = jnp.zeros(w1a_ref.shape, bf16)
    w2a_ref[...] = jnp.zeros(w2a_ref.shape, bf16)

    for c in range(6):
        for q in range(2):
            s = sum(w1_ref[c * 25 + t] * e1_ref[q, t] for t in range(25))
            sw = s.astype(bf16)
            for p in range(2):
                base = (c * 4 + p * 2 + q) * 144
                for y in range(12):
                    col = (2 * y + p) * 28
                    w1a_ref[pl.ds(base + y * 12, 12), col:col + 140] = sw
        w1a_ref[pl.ds(c * 576, 576), 784:785] = jnp.full(
            (576, 1), b1_ref[c], bf16)

    for co in range(12):
        for cin in range(6):
            for q in range(2):
                s = sum(w2_ref[(co * 6 + cin) * 25 + t] * e2_ref[q, t]
                        for t in range(25))
                sw = s.astype(bf16)
                for p in range(2):
                    base = (co * 4 + p * 2 + q) * 16
                    for y in range(4):
                        col = cin * 144 + (2 * y + p) * 12
                        w2a_ref[pl.ds(base + y * 4, 4), col:col + 60] = sw
        w2a_ref[pl.ds(co * 64, 64), 864:865] = jnp.full(
            (64, 1), b2_ref[co], bf16)


def _lenet_body(xb_ref, w1_ref, b1_ref, w2_ref, b2_ref, e1_ref, e2_ref,
                w1f_ref, w2f_ref, w3f_ref, out_ref, w1a_ref, w2a_ref):
    f32 = jnp.float32
    bf16 = jnp.bfloat16

    @pl.when(pl.program_id(0) == 0)
    def _():
        _build_weights(w1_ref, b1_ref, w2_ref, b2_ref, e1_ref, e2_ref,
                       w1a_ref, w2a_ref)

    ones_row = jnp.ones((1, _B), f32)

    xaug = jnp.concatenate([xb_ref[...], ones_row], axis=0).astype(bf16)
    h1 = jnp.dot(w1a_ref[...], xaug, preferred_element_type=f32)
    h1 = jnp.maximum(h1, 0.0)

    p1 = jnp.concatenate(
        [jnp.maximum(
            jnp.maximum(h1[(c * 4 + 0) * 144:(c * 4 + 1) * 144],
                        h1[(c * 4 + 1) * 144:(c * 4 + 2) * 144]),
            jnp.maximum(h1[(c * 4 + 2) * 144:(c * 4 + 3) * 144],
                        h1[(c * 4 + 3) * 144:(c * 4 + 4) * 144]))
         for c in range(6)] + [ones_row], axis=0).astype(bf16)

    h2 = jnp.dot(w2a_ref[...], p1, preferred_element_type=f32)
    h2 = jnp.maximum(h2, 0.0)

    p2 = jnp.concatenate(
        [jnp.maximum(
            jnp.maximum(h2[(c * 4 + 0) * 16:(c * 4 + 1) * 16],
                        h2[(c * 4 + 1) * 16:(c * 4 + 2) * 16]),
            jnp.maximum(h2[(c * 4 + 2) * 16:(c * 4 + 3) * 16],
                        h2[(c * 4 + 3) * 16:(c * 4 + 4) * 16]))
         for c in range(12)] + [ones_row], axis=0)

    h3 = jnp.maximum(jnp.dot(w1f_ref[...], p2, preferred_element_type=f32), 0.0)
    h3 = jnp.concatenate([h3, ones_row], axis=0)
    h4 = jnp.maximum(jnp.dot(w2f_ref[...], h3, preferred_element_type=f32), 0.0)
    h4 = jnp.concatenate([h4, ones_row], axis=0)
    out_ref[...] = lax.dot_general(h4, w3f_ref[...], (((0,), (1,)), ((), ())),
                                   preferred_element_type=f32)


def kernel(x, conv1_w, conv1_b, conv2_w, conv2_b,
           fc1_w, fc1_b, fc2_w, fc2_b, out_w, out_b):
    f32 = jnp.float32
    n = x.shape[0]
    nt = -(-n // _B)
    npad = nt * _B

    x2 = x.astype(f32)[:, 0].transpose(1, 2, 0).reshape(784, n)
    if npad != n:
        x2 = jnp.pad(x2, ((0, 0), (0, npad - n)))

    w1s = conv1_w.astype(f32).reshape(150)
    w2s = conv2_w.astype(f32).reshape(1800)
    b1s = conv1_b.astype(f32)
    b2s = conv2_b.astype(f32)

    w1f = jnp.concatenate([fc1_w.astype(f32), fc1_b.astype(f32)[:, None]], axis=1)
    w2f = jnp.concatenate([fc2_w.astype(f32), fc2_b.astype(f32)[:, None]], axis=1)
    w3f = jnp.concatenate([out_w.astype(f32), out_b.astype(f32)[:, None]], axis=1)

    smem = pl.BlockSpec(memory_space=pltpu.MemorySpace.SMEM)

    def whole(a):
        return pl.BlockSpec(a.shape, lambda i: (0,) * a.ndim)

    e1 = jnp.asarray(_E1)
    e2 = jnp.asarray(_E2)

    out = pl.pallas_call(
        _lenet_body,
        out_shape=jax.ShapeDtypeStruct((npad, 10), f32),
        grid=(nt,),
        in_specs=[
            pl.BlockSpec((784, _B), lambda i: (0, i)),
            smem, smem, smem, smem,
            whole(e1), whole(e2),
            whole(w1f), whole(w2f), whole(w3f),
        ],
        out_specs=pl.BlockSpec((_B, 10), lambda i: (i, 0)),
        scratch_shapes=[
            pltpu.VMEM((3456, 785), jnp.bfloat16),
            pltpu.VMEM((768, 865), jnp.bfloat16),
        ],
        compiler_params=pltpu.CompilerParams(
            dimension_semantics=("arbitrary",),
            vmem_limit_bytes=48 * 1024 * 1024,
        ),
    )(x2, w1s, b1s, w2s, b2s, e1, e2, w1f, w2f, w3f)

    return out[:n]


# speedup vs baseline: 2.1194x; 1.0339x over previous
import numpy as np
import jax
import jax.numpy as jnp
from jax import lax
from jax.experimental import pallas as pl
from jax.experimental.pallas import tpu as pltpu

_B = 1024


def _masks1():
    e = np.zeros((2, 25, 12, 140), np.float32)
    for q in range(2):
        for k in range(5):
            for l in range(5):
                for x in range(12):
                    e[q, k * 5 + l, x, k * 28 + 2 * x + q + l] = 1.0
    return e


def _masks2():
    e = np.zeros((2, 25, 4, 60), np.float32)
    for q in range(2):
        for k in range(5):
            for l in range(5):
                for x in range(4):
                    e[q, k * 5 + l, x, k * 12 + 2 * x + q + l] = 1.0
    return e


_E1 = _masks1()
_E2 = _masks2()


def _build_weights(w1_ref, b1_ref, w2_ref, b2_ref, e1_ref, e2_ref,
                   w1a_ref, w2a_ref):
    f32 = jnp.float32
    bf16 = jnp.bfloat16

    w1a_ref[...] = jnp.zeros(w1a_ref.shape, bf16)
    w2a_ref[...] = jnp.zeros(w2a_ref.shape, bf16)

    for c in range(6):
        for q in range(2):
            s = sum(w1_ref[c * 25 + t] * e1_ref[q, t] for t in range(25))
            sw = s.astype(bf16)
            for p in range(2):
                base = (c * 4 + p * 2 + q) * 144
                for y in range(12):
                    col = (2 * y + p) * 28
                    w1a_ref[pl.ds(base + y * 12, 12), col:col + 140] = sw
        w1a_ref[pl.ds(c * 576, 576), 784:785] = jnp.full(
            (576, 1), b1_ref[c], bf16)

    for co in range(12):
        for cin in range(6):
            for q in range(2):
                s = sum(w2_ref[(co * 6 + cin) * 25 + t] * e2_ref[q, t]
                        for t in range(25))
                sw = s.astype(bf16)
                for p in range(2):
                    base = (co * 4 + p * 2 + q) * 16
                    for y in range(4):
                        col = cin * 144 + (2 * y + p) * 12
                        w2a_ref[pl.ds(base + y * 4, 4), col:col + 60] = sw
        w2a_ref[pl.ds(co * 64, 64), 864:865] = jnp.full(
            (64, 1), b2_ref[co], bf16)


def _lenet_body(xb_ref, w1_ref, b1_ref, w2_ref, b2_ref, e1_ref, e2_ref,
                w1f_ref, w2f_ref, w3f_ref, out_ref, w1a_ref, w2a_ref):
    f32 = jnp.float32
    bf16 = jnp.bfloat16

    @pl.when(pl.program_id(0) == 0)
    def _():
        _build_weights(w1_ref, b1_ref, w2_ref, b2_ref, e1_ref, e2_ref,
                       w1a_ref, w2a_ref)

    ones_row = jnp.ones((1, _B), f32)

    xaug = jnp.concatenate([xb_ref[...], ones_row], axis=0).astype(bf16)
    h1 = jnp.dot(w1a_ref[...], xaug, preferred_element_type=f32)
    h1 = jnp.maximum(h1, 0.0)

    p1 = jnp.concatenate(
        [jnp.maximum(
            jnp.maximum(h1[(c * 4 + 0) * 144:(c * 4 + 1) * 144],
                        h1[(c * 4 + 1) * 144:(c * 4 + 2) * 144]),
            jnp.maximum(h1[(c * 4 + 2) * 144:(c * 4 + 3) * 144],
                        h1[(c * 4 + 3) * 144:(c * 4 + 4) * 144]))
         for c in range(6)] + [ones_row], axis=0).astype(bf16)

    h2 = jnp.dot(w2a_ref[...], p1, preferred_element_type=f32)
    h2 = jnp.maximum(h2, 0.0)

    p2 = jnp.concatenate(
        [jnp.maximum(
            jnp.maximum(h2[(c * 4 + 0) * 16:(c * 4 + 1) * 16],
                        h2[(c * 4 + 1) * 16:(c * 4 + 2) * 16]),
            jnp.maximum(h2[(c * 4 + 2) * 16:(c * 4 + 3) * 16],
                        h2[(c * 4 + 3) * 16:(c * 4 + 4) * 16]))
         for c in range(12)] + [ones_row], axis=0)

    h3 = jnp.maximum(jnp.dot(w1f_ref[...], p2, preferred_element_type=f32), 0.0)
    h3 = jnp.concatenate([h3, ones_row], axis=0)
    h4 = jnp.maximum(jnp.dot(w2f_ref[...], h3, preferred_element_type=f32), 0.0)
    h4 = jnp.concatenate([h4, ones_row], axis=0)
    out_ref[...] = lax.dot_general(h4, w3f_ref[...], (((0,), (1,)), ((), ())),
                                   preferred_element_type=f32)


def kernel(x, conv1_w, conv1_b, conv2_w, conv2_b,
           fc1_w, fc1_b, fc2_w, fc2_b, out_w, out_b):
    f32 = jnp.float32
    n = x.shape[0]
    nt = -(-n // _B)
    npad = nt * _B

    x2 = x.astype(f32)[:, 0].transpose(1, 2, 0).reshape(784, n)
    if npad != n:
        x2 = jnp.pad(x2, ((0, 0), (0, npad - n)))

    w1s = conv1_w.astype(f32).reshape(150)
    w2s = conv2_w.astype(f32).reshape(1800)
    b1s = conv1_b.astype(f32)
    b2s = conv2_b.astype(f32)

    w1f = jnp.concatenate([fc1_w.astype(f32), fc1_b.astype(f32)[:, None]], axis=1)
    w2f = jnp.concatenate([fc2_w.astype(f32), fc2_b.astype(f32)[:, None]], axis=1)
    w3f = jnp.concatenate([out_w.astype(f32), out_b.astype(f32)[:, None]], axis=1)

    smem = pl.BlockSpec(memory_space=pltpu.MemorySpace.SMEM)

    def whole(a):
        return pl.BlockSpec(a.shape, lambda i: (0,) * a.ndim)

    e1 = jnp.asarray(_E1)
    e2 = jnp.asarray(_E2)

    out = pl.pallas_call(
        _lenet_body,
        out_shape=jax.ShapeDtypeStruct((npad, 10), f32),
        grid=(nt,),
        in_specs=[
            pl.BlockSpec((784, _B), lambda i: (0, i)),
            smem, smem, smem, smem,
            whole(e1), whole(e2),
            whole(w1f), whole(w2f), whole(w3f),
        ],
        out_specs=pl.BlockSpec((_B, 10), lambda i: (i, 0)),
        scratch_shapes=[
            pltpu.VMEM((3456, 785), jnp.bfloat16),
            pltpu.VMEM((768, 865), jnp.bfloat16),
        ],
        compiler_params=pltpu.CompilerParams(
            dimension_semantics=("arbitrary",),
            vmem_limit_bytes=48 * 1024 * 1024,
        ),
    )(x2, w1s, b1s, w2s, b2s, e1, e2, w1f, w2f, w3f)

    return out[:n]
